# Initial kernel scaffold; baseline (speedup 1.0000x reference)
import jax
import jax.numpy as jnp
from jax.experimental import pallas as pl


def kernel(feats, adj, W1, b1, W2, b2, Wp, bp):
    raise NotImplementedError("write your pallas kernel here")


# speedup vs baseline: 1.0697x; 1.0697x over previous
import jax
import jax.numpy as jnp
from jax.experimental import pallas as pl
from jax.experimental.pallas import tpu as pltpu


def _pass_a(adj_ref, feats_ref, w1_ref, v1_ref):
    adj = adj_ref[...]
    deg = jnp.sum(adj, axis=1)
    dinv = jax.lax.rsqrt(deg + 1e-9)
    xw = jnp.dot(feats_ref[...], w1_ref[...], preferred_element_type=jnp.float32)
    v1_ref[...] = dinv[:, None] * xw


def _pass_b(adj_ref, v1_ref, b1_ref, w2_ref, v2_ref):
    adj = adj_ref[...]
    t = jnp.dot(adj, v1_ref[...], preferred_element_type=jnp.float32)
    deg = jnp.sum(adj, axis=1)
    dinv = jax.lax.rsqrt(deg + 1e-9)
    h = jnp.maximum(dinv[:, None] * t + b1_ref[...], 0.0)
    v2_ref[...] = dinv[:, None] * jnp.dot(
        h, w2_ref[...], preferred_element_type=jnp.float32)


def _pass_c(adj_ref, v2_ref, b2_ref, wp_ref, bp_ref, out_ref):
    adj = adj_ref[...]
    t = jnp.dot(adj, v2_ref[...], preferred_element_type=jnp.float32)
    deg = jnp.sum(adj, axis=1)
    dinv = jax.lax.rsqrt(deg + 1e-9)
    h = jnp.maximum(dinv[:, None] * t + b2_ref[...], 0.0)
    out_ref[...] = jnp.dot(
        h, wp_ref[...], preferred_element_type=jnp.float32) + bp_ref[...]


def kernel(feats, adj, W1, b1, W2, b2, Wp, bp):
    n, d = feats.shape
    h = W1.shape[1]
    bi = 400
    grid = (n // bi,)

    b1r = b1.reshape(1, h)
    b2r = b2.reshape(1, h)
    bpr = bp.reshape(1, 1)

    full = lambda *shape: pl.BlockSpec(shape, lambda i: (0,) * len(shape))
    rows = lambda *shape: pl.BlockSpec(shape, lambda i: (i,) + (0,) * (len(shape) - 1))

    params = pltpu.CompilerParams(dimension_semantics=("arbitrary",))

    v1 = pl.pallas_call(
        _pass_a,
        grid=grid,
        in_specs=[rows(bi, n), rows(bi, d), full(d, h)],
        out_specs=rows(bi, h),
        out_shape=jax.ShapeDtypeStruct((n, h), jnp.float32),
        compiler_params=params,
    )(adj, feats, W1)

    v2 = pl.pallas_call(
        _pass_b,
        grid=grid,
        in_specs=[rows(bi, n), full(n, h), full(1, h), full(h, h)],
        out_specs=rows(bi, h),
        out_shape=jax.ShapeDtypeStruct((n, h), jnp.float32),
        compiler_params=params,
    )(adj, v1, b1r, W2)

    out = pl.pallas_call(
        _pass_c,
        grid=grid,
        in_specs=[rows(bi, n), full(n, h), full(1, h), full(h, 1), full(1, 1)],
        out_specs=rows(bi, 1),
        out_shape=jax.ShapeDtypeStruct((n, 1), jnp.float32),
        compiler_params=params,
    )(adj, v2, b2r, Wp, bpr)

    return out


# speedup vs baseline: 1.1192x; 1.0462x over previous
import jax
import jax.numpy as jnp
from jax.experimental import pallas as pl
from jax.experimental.pallas import tpu as pltpu


def _pass_a(adj_ref, feats_ref, w1_ref, v1_ref):
    adj = adj_ref[...]
    deg = jnp.sum(adj, axis=1)
    dinv = jax.lax.rsqrt(deg + 1e-9)
    xw = jnp.dot(feats_ref[...].astype(jnp.bfloat16), w1_ref[...].astype(jnp.bfloat16),
                 preferred_element_type=jnp.float32)
    v1_ref[...] = (dinv[:, None] * xw).astype(jnp.bfloat16)


def _pass_b(adj_ref, v1_ref, b1_ref, w2_ref, v2_ref):
    adj = adj_ref[...]
    t = jnp.dot(adj.astype(jnp.bfloat16), v1_ref[...],
                preferred_element_type=jnp.float32)
    deg = jnp.sum(adj, axis=1)
    dinv = jax.lax.rsqrt(deg + 1e-9)
    h = jnp.maximum(dinv[:, None] * t + b1_ref[...], 0.0)
    v2_ref[...] = (dinv[:, None] * jnp.dot(
        h.astype(jnp.bfloat16), w2_ref[...].astype(jnp.bfloat16),
        preferred_element_type=jnp.float32)).astype(jnp.bfloat16)


def _pass_c(adj_ref, v2_ref, b2_ref, wp_ref, bp_ref, out_ref):
    adj = adj_ref[...]
    t = jnp.dot(adj.astype(jnp.bfloat16), v2_ref[...],
                preferred_element_type=jnp.float32)
    deg = jnp.sum(adj, axis=1)
    dinv = jax.lax.rsqrt(deg + 1e-9)
    h = jnp.maximum(dinv[:, None] * t + b2_ref[...], 0.0)
    out_ref[...] = jnp.dot(
        h, wp_ref[...], preferred_element_type=jnp.float32) + bp_ref[...]


def kernel(feats, adj, W1, b1, W2, b2, Wp, bp):
    n, d = feats.shape
    h = W1.shape[1]
    bi = 400
    grid = (n // bi,)

    b1r = b1.reshape(1, h)
    b2r = b2.reshape(1, h)
    bpr = bp.reshape(1, 1)

    full = lambda *shape: pl.BlockSpec(shape, lambda i: (0,) * len(shape))
    rows = lambda *shape: pl.BlockSpec(shape, lambda i: (i,) + (0,) * (len(shape) - 1))

    params = pltpu.CompilerParams(dimension_semantics=("arbitrary",))

    v1 = pl.pallas_call(
        _pass_a,
        grid=grid,
        in_specs=[rows(bi, n), rows(bi, d), full(d, h)],
        out_specs=rows(bi, h),
        out_shape=jax.ShapeDtypeStruct((n, h), jnp.bfloat16),
        compiler_params=params,
    )(adj, feats, W1)

    v2 = pl.pallas_call(
        _pass_b,
        grid=grid,
        in_specs=[rows(bi, n), full(n, h), full(1, h), full(h, h)],
        out_specs=rows(bi, h),
        out_shape=jax.ShapeDtypeStruct((n, h), jnp.bfloat16),
        compiler_params=params,
    )(adj, v1, b1r, W2)

    out = pl.pallas_call(
        _pass_c,
        grid=grid,
        in_specs=[rows(bi, n), full(n, h), full(1, h), full(h, 1), full(1, 1)],
        out_specs=rows(bi, 1),
        out_shape=jax.ShapeDtypeStruct((n, 1), jnp.float32),
        compiler_params=params,
    )(adj, v2, b2r, Wp, bpr)

    return out
